# baseline (speedup 1.0000x reference)
import functools

import jax
import jax.numpy as jnp
from jax import lax
from jax.experimental import pallas as pl
from jax.experimental.pallas import tpu as pltpu
from jax.experimental.pallas import tpu_sc as plsc

B, F, V, E = 4096, 26, 100000, 64
D_IN = F * E
H = 400
BF = B * F
NW = 32
PERW = BF // NW
CHUNK = 128
NCH = PERW // CHUNK
BB = 512
NB = B // BB
EPS = 1e-3


def _sc_gather(idx3, e_table, w_table):
    mesh = plsc.VectorSubcoreMesh(core_axis_name="c", subcore_axis_name="s")

    @functools.partial(
        pl.kernel,
        out_type=[
            jax.ShapeDtypeStruct((BF, E), jnp.float32),
            jax.ShapeDtypeStruct((BF, 1), jnp.float32),
        ],
        mesh=mesh,
        compiler_params=pltpu.CompilerParams(use_tc_tiling_on_sc=False),
        scratch_types=[
            pltpu.VMEM((NCH, CHUNK), jnp.int32),
            pltpu.VMEM((CHUNK, E), jnp.float32),
            pltpu.VMEM((CHUNK, 1), jnp.float32),
            pltpu.SemaphoreType.DMA,
            pltpu.SemaphoreType.DMA,
        ],
    )
    def k(idx_hbm, etab_hbm, wtab_hbm, emb_out, w_out,
          idx_v, rows0, wrow, sem0, semw):
        wid = lax.axis_index("s") * 2 + lax.axis_index("c")
        base = wid * PERW
        pltpu.sync_copy(idx_hbm.at[wid], idx_v)

        @pl.loop(0, NCH)
        def _(c):
            pltpu.async_copy(etab_hbm.at[idx_v.at[c]], rows0, sem0)
            pltpu.async_copy(wtab_hbm.at[idx_v.at[c]], wrow, semw)
            pltpu.make_async_copy(etab_hbm.at[idx_v.at[c]], rows0, sem0).wait()
            pltpu.sync_copy(rows0, emb_out.at[pl.ds(base + c * CHUNK, CHUNK)])
            pltpu.make_async_copy(wtab_hbm.at[idx_v.at[c]], wrow, semw).wait()
            pltpu.sync_copy(wrow, w_out.at[pl.ds(base + c * CHUNK, CHUNK)])

    return k(idx3, e_table, w_table)


def _tc1_body(emb_ref, val_ref, wg_ref, w0_ref, b0_ref,
              h0_ref, lin_ref, fm_ref, stats_ref):
    i = pl.program_id(0)
    emb = emb_ref[...]
    val = val_ref[...]
    h0 = jnp.dot(emb, w0_ref[...], preferred_element_type=jnp.float32)
    h0 = h0 + b0_ref[...]
    h0_ref[...] = h0
    lin_ref[...] = wg_ref[...] * val
    s = jnp.zeros((BB, E), jnp.float32)
    s2 = jnp.zeros((BB, E), jnp.float32)
    for f in range(F):
        t = emb[:, f * E:(f + 1) * E] * val[:, f:f + 1]
        s = s + t
        s2 = s2 + t * t
    fm_ref[...] = 0.5 * (s * s - s2)
    ps = jnp.concatenate(
        [jnp.sum(h0, axis=0, keepdims=True),
         jnp.sum(h0 * h0, axis=0, keepdims=True)], axis=0)

    @pl.when(i == 0)
    def _():
        stats_ref[...] = jnp.zeros_like(stats_ref)

    stats_ref[...] += ps


def _tc2_body(h0_ref, stats_ref, g0_ref, bt0_ref, w1_ref, b1_ref,
              h1_ref, stats2_ref):
    i = pl.program_id(0)
    s = stats_ref[...]
    m = s[0:1, :] * (1.0 / B)
    v = s[1:2, :] * (1.0 / B) - m * m
    inv = g0_ref[...] * lax.rsqrt(v + EPS)
    a = jnp.maximum((h0_ref[...] - m) * inv + bt0_ref[...], 0.0)
    h1 = jnp.dot(a, w1_ref[...], preferred_element_type=jnp.float32)
    h1 = h1 + b1_ref[...]
    h1_ref[...] = h1
    ps = jnp.concatenate(
        [jnp.sum(h1, axis=0, keepdims=True),
         jnp.sum(h1 * h1, axis=0, keepdims=True)], axis=0)

    @pl.when(i == 0)
    def _():
        stats2_ref[...] = jnp.zeros_like(stats2_ref)

    stats2_ref[...] += ps


def _tc3_body(h1_ref, stats2_ref, g1_ref, bt1_ref, lin_ref, fm_ref,
              wfca_ref, wfcb_ref, wfcc_ref, bfc_ref, out_ref):
    s = stats2_ref[...]
    m = s[0:1, :] * (1.0 / B)
    v = s[1:2, :] * (1.0 / B) - m * m
    inv = g1_ref[...] * lax.rsqrt(v + EPS)
    y = jnp.maximum((h1_ref[...] - m) * inv + bt1_ref[...], 0.0)
    logit = (jnp.sum(lin_ref[...] * wfca_ref[...], axis=1, keepdims=True)
             + jnp.sum(fm_ref[...] * wfcb_ref[...], axis=1, keepdims=True)
             + jnp.dot(y, wfcc_ref[...], preferred_element_type=jnp.float32)
             + bfc_ref[...])
    out_ref[...] = jax.nn.sigmoid(logit)


def _tc1(emb, val, wg, W0, b0):
    return pl.pallas_call(
        _tc1_body,
        grid=(NB,),
        in_specs=[
            pl.BlockSpec((BB, D_IN), lambda i: (i, 0)),
            pl.BlockSpec((BB, F), lambda i: (i, 0)),
            pl.BlockSpec((BB, F), lambda i: (i, 0)),
            pl.BlockSpec((D_IN, H), lambda i: (0, 0)),
            pl.BlockSpec((1, H), lambda i: (0, 0)),
        ],
        out_specs=[
            pl.BlockSpec((BB, H), lambda i: (i, 0)),
            pl.BlockSpec((BB, F), lambda i: (i, 0)),
            pl.BlockSpec((BB, E), lambda i: (i, 0)),
            pl.BlockSpec((2, H), lambda i: (0, 0)),
        ],
        out_shape=[
            jax.ShapeDtypeStruct((B, H), jnp.float32),
            jax.ShapeDtypeStruct((B, F), jnp.float32),
            jax.ShapeDtypeStruct((B, E), jnp.float32),
            jax.ShapeDtypeStruct((2, H), jnp.float32),
        ],
    )(emb, val, wg, W0, b0)


def _tc2(h0, stats, g0, bt0, W1, b1):
    return pl.pallas_call(
        _tc2_body,
        grid=(NB,),
        in_specs=[
            pl.BlockSpec((BB, H), lambda i: (i, 0)),
            pl.BlockSpec((2, H), lambda i: (0, 0)),
            pl.BlockSpec((1, H), lambda i: (0, 0)),
            pl.BlockSpec((1, H), lambda i: (0, 0)),
            pl.BlockSpec((H, H), lambda i: (0, 0)),
            pl.BlockSpec((1, H), lambda i: (0, 0)),
        ],
        out_specs=[
            pl.BlockSpec((BB, H), lambda i: (i, 0)),
            pl.BlockSpec((2, H), lambda i: (0, 0)),
        ],
        out_shape=[
            jax.ShapeDtypeStruct((B, H), jnp.float32),
            jax.ShapeDtypeStruct((2, H), jnp.float32),
        ],
    )(h0, stats, g0, bt0, W1, b1)


def _tc3(h1, stats2, g1, bt1, lin, fm, wfca, wfcb, wfcc, bfc):
    return pl.pallas_call(
        _tc3_body,
        grid=(NB,),
        in_specs=[
            pl.BlockSpec((BB, H), lambda i: (i, 0)),
            pl.BlockSpec((2, H), lambda i: (0, 0)),
            pl.BlockSpec((1, H), lambda i: (0, 0)),
            pl.BlockSpec((1, H), lambda i: (0, 0)),
            pl.BlockSpec((BB, F), lambda i: (i, 0)),
            pl.BlockSpec((BB, E), lambda i: (i, 0)),
            pl.BlockSpec((1, F), lambda i: (0, 0)),
            pl.BlockSpec((1, E), lambda i: (0, 0)),
            pl.BlockSpec((H, 1), lambda i: (0, 0)),
            pl.BlockSpec((1, 1), lambda i: (0, 0)),
        ],
        out_specs=pl.BlockSpec((BB, 1), lambda i: (i, 0)),
        out_shape=jax.ShapeDtypeStruct((B, 1), jnp.float32),
    )(h1, stats2, g1, bt1, lin, fm, wfca, wfcb, wfcc, bfc)


def kernel(feat_idx, feat_val, w_table, e_table,
           W0, b0, g0, bt0, W1, b1, g1, bt1, Wfc, bfc):
    idx3 = feat_idx.astype(jnp.int32).reshape(NW, NCH, CHUNK)
    emb_flat, wg_flat = _sc_gather(idx3, e_table, w_table)
    emb = emb_flat.reshape(B, D_IN)
    wg = wg_flat.reshape(B, F)
    val = feat_val.astype(jnp.float32)

    h0, lin, fm, stats = _tc1(emb, val, wg, W0, b0.reshape(1, H))
    h1, stats2 = _tc2(h0, stats, g0.reshape(1, H), bt0.reshape(1, H),
                      W1, b1.reshape(1, H))
    out = _tc3(h1, stats2, g1.reshape(1, H), bt1.reshape(1, H), lin, fm,
               Wfc[:F].reshape(1, F), Wfc[F:F + E].reshape(1, E),
               Wfc[F + E:], bfc.reshape(1, 1))
    return out


# speedup vs baseline: 1.0693x; 1.0693x over previous
import functools

import jax
import jax.numpy as jnp
from jax import lax
from jax.experimental import pallas as pl
from jax.experimental.pallas import tpu as pltpu
from jax.experimental.pallas import tpu_sc as plsc

B, F, V, E = 4096, 26, 100000, 64
D_IN = F * E
H = 400
BF = B * F
NW = 32
PERW = BF // NW
CHUNK = 128
NCH = PERW // CHUNK
BB = 512
NB = B // BB
EPS = 1e-3


def _sc_gather(idx_flat, e_table, w_table):
    mesh = plsc.VectorSubcoreMesh(core_axis_name="c", subcore_axis_name="s")

    @functools.partial(
        pl.kernel,
        out_type=[
            jax.ShapeDtypeStruct((BF, E), jnp.float32),
            jax.ShapeDtypeStruct((BF, 1), jnp.float32),
        ],
        mesh=mesh,
        compiler_params=pltpu.CompilerParams(use_tc_tiling_on_sc=False),
        scratch_types=[
            pltpu.VMEM((PERW,), jnp.int32),
            pltpu.VMEM((CHUNK, E), jnp.float32),
            pltpu.VMEM((CHUNK, 1), jnp.float32),
            pltpu.SemaphoreType.DMA,
            pltpu.SemaphoreType.DMA,
        ],
    )
    def k(idx_hbm, etab_hbm, wtab_hbm, emb_out, w_out,
          idx_v, rows0, wrow, sem0, semw):
        wid = lax.axis_index("s") * 2 + lax.axis_index("c")
        base = wid * PERW
        pltpu.sync_copy(idx_hbm.at[pl.ds(base, PERW)], idx_v)

        @pl.loop(0, NCH)
        def _(c):
            ix = idx_v.at[pl.ds(c * CHUNK, CHUNK)]
            pltpu.async_copy(etab_hbm.at[ix], rows0, sem0)
            pltpu.async_copy(wtab_hbm.at[ix], wrow, semw)
            pltpu.make_async_copy(etab_hbm.at[ix], rows0, sem0).wait()
            pltpu.sync_copy(rows0, emb_out.at[pl.ds(base + c * CHUNK, CHUNK)])
            pltpu.make_async_copy(wtab_hbm.at[ix], wrow, semw).wait()
            pltpu.sync_copy(wrow, w_out.at[pl.ds(base + c * CHUNK, CHUNK)])

    return k(idx_flat, e_table, w_table)


def _fused_body(emb_ref, val_ref, wg_ref, w0_ref, w1_ref, bias_ref,
                r_ref, s_ref, wfca_ref, wfcb_ref, wfcc_ref, bfc_ref,
                out_ref, h0_s, h1_s, lin_s, fm_s, st0_s, st1_s):
    p = pl.program_id(0)
    i = pl.program_id(1)

    @pl.when(p == 0)
    def _():
        emb = emb_ref[...]
        val = val_ref[...]
        h0 = jnp.dot(emb, w0_ref[...], preferred_element_type=jnp.float32)
        h0 = h0 + bias_ref[0:1, :H]
        h0_s[pl.ds(i * BB, BB), :] = h0
        lin_s[pl.ds(i * BB, BB), :] = wg_ref[...] * val
        vexp = jnp.dot(val, r_ref[...], preferred_element_type=jnp.float32)
        t = emb * vexp
        s = jnp.dot(t, s_ref[...], preferred_element_type=jnp.float32)
        s2 = jnp.dot(t * t, s_ref[...], preferred_element_type=jnp.float32)
        fm_s[pl.ds(i * BB, BB), :] = 0.5 * (s * s - s2)
        ps = jnp.concatenate(
            [jnp.sum(h0, axis=0, keepdims=True),
             jnp.sum(h0 * h0, axis=0, keepdims=True)], axis=0)

        @pl.when(i == 0)
        def _():
            st0_s[...] = jnp.zeros_like(st0_s)

        st0_s[...] += ps

    @pl.when(p == 1)
    def _():
        st = st0_s[...]
        m = st[0:1, :] * (1.0 / B)
        v = st[1:2, :] * (1.0 / B) - m * m
        inv = bias_ref[1:2, :H] * lax.rsqrt(v + EPS)
        h0 = h0_s[pl.ds(i * BB, BB), :]
        a = jnp.maximum((h0 - m) * inv + bias_ref[2:3, :H], 0.0)
        h1 = jnp.dot(a, w1_ref[...], preferred_element_type=jnp.float32)
        h1 = h1 + bias_ref[3:4, :H]
        h1_s[pl.ds(i * BB, BB), :] = h1
        ps = jnp.concatenate(
            [jnp.sum(h1, axis=0, keepdims=True),
             jnp.sum(h1 * h1, axis=0, keepdims=True)], axis=0)

        @pl.when(i == 0)
        def _():
            st1_s[...] = jnp.zeros_like(st1_s)

        st1_s[...] += ps

    @pl.when(p == 2)
    def _():
        st = st1_s[...]
        m = st[0:1, :] * (1.0 / B)
        v = st[1:2, :] * (1.0 / B) - m * m
        inv = bias_ref[4:5, :H] * lax.rsqrt(v + EPS)
        h1 = h1_s[pl.ds(i * BB, BB), :]
        y = jnp.maximum((h1 - m) * inv + bias_ref[5:6, :H], 0.0)
        logit = (jnp.sum(lin_s[pl.ds(i * BB, BB), :] * wfca_ref[...],
                         axis=1, keepdims=True)
                 + jnp.sum(fm_s[pl.ds(i * BB, BB), :] * wfcb_ref[...],
                           axis=1, keepdims=True)
                 + jnp.dot(y, wfcc_ref[...],
                           preferred_element_type=jnp.float32)
                 + bfc_ref[...])
        out_ref[...] = jax.nn.sigmoid(logit)


def _tc_fused(emb, val, wg, W0, W1, bias6, R, S, wfca, wfcb, wfcc, bfc):
    def eb(p, i):
        return (jnp.where(p == 0, i, NB - 1), 0)

    def cst(p, i):
        return (0, 0)

    return pl.pallas_call(
        _fused_body,
        grid=(3, NB),
        in_specs=[
            pl.BlockSpec((BB, D_IN), eb),
            pl.BlockSpec((BB, F), eb),
            pl.BlockSpec((BB, F), eb),
            pl.BlockSpec((D_IN, H), cst),
            pl.BlockSpec((H, H), cst),
            pl.BlockSpec((6, H), cst),
            pl.BlockSpec((F, D_IN), cst),
            pl.BlockSpec((D_IN, E), cst),
            pl.BlockSpec((1, F), cst),
            pl.BlockSpec((1, E), cst),
            pl.BlockSpec((H, 1), cst),
            pl.BlockSpec((1, 1), cst),
        ],
        out_specs=pl.BlockSpec((BB, 1), lambda p, i: (jnp.where(p == 2, i, 0), 0)),
        out_shape=jax.ShapeDtypeStruct((B, 1), jnp.float32),
        scratch_shapes=[
            pltpu.VMEM((B, H), jnp.float32),
            pltpu.VMEM((B, H), jnp.float32),
            pltpu.VMEM((B, F), jnp.float32),
            pltpu.VMEM((B, E), jnp.float32),
            pltpu.VMEM((2, H), jnp.float32),
            pltpu.VMEM((2, H), jnp.float32),
        ],
    )(emb, val, wg, W0, W1, bias6, R, S, wfca, wfcb, wfcc, bfc)


def kernel(feat_idx, feat_val, w_table, e_table,
           W0, b0, g0, bt0, W1, b1, g1, bt1, Wfc, bfc):
    idx_flat = feat_idx.astype(jnp.int32).reshape(BF)
    emb_flat, wg_flat = _sc_gather(idx_flat, e_table, w_table)
    emb = emb_flat.reshape(B, D_IN)
    wg = wg_flat.reshape(B, F)
    val = feat_val.astype(jnp.float32)

    bias6 = jnp.stack([b0, g0, bt0, b1, g1, bt1], axis=0)
    R = jnp.repeat(jnp.eye(F, dtype=jnp.float32), E, axis=1)
    S = jnp.tile(jnp.eye(E, dtype=jnp.float32), (F, 1))

    return _tc_fused(emb, val, wg, W0, W1, bias6, R, S,
                     Wfc[:F].reshape(1, F), Wfc[F:F + E].reshape(1, E),
                     Wfc[F + E:], bfc.reshape(1, 1))


# speedup vs baseline: 1.8740x; 1.7525x over previous
import functools

import jax
import jax.numpy as jnp
from jax import lax
from jax.experimental import pallas as pl
from jax.experimental.pallas import tpu as pltpu
from jax.experimental.pallas import tpu_sc as plsc

B, F, V, E = 4096, 26, 100000, 64
D_IN = F * E
H = 400
BF = B * F
NW = 32
PERW = BF // NW
CHUNK = 128
NCH = PERW // CHUNK
BB = 512
NB = B // BB
EPS = 1e-3


def _sc_gather(idx_flat, e_table, w_flat):
    mesh = plsc.VectorSubcoreMesh(core_axis_name="c", subcore_axis_name="s")

    @functools.partial(
        pl.kernel,
        out_type=[
            jax.ShapeDtypeStruct((BF, E), jnp.float32),
            jax.ShapeDtypeStruct((BF,), jnp.float32),
        ],
        mesh=mesh,
        compiler_params=pltpu.CompilerParams(use_tc_tiling_on_sc=False,
                                             needs_layout_passes=False),
        scratch_types=[
            pltpu.VMEM((PERW,), jnp.int32),
            pltpu.VMEM((CHUNK, E), jnp.float32),
            pltpu.VMEM((CHUNK, E), jnp.float32),
            pltpu.VMEM((V,), jnp.float32),
            pltpu.VMEM((PERW,), jnp.float32),
            pltpu.SemaphoreType.DMA,
            pltpu.SemaphoreType.DMA,
            pltpu.SemaphoreType.DMA,
        ],
    )
    def k(idx_hbm, etab_hbm, wtab_hbm, emb_out, w_out,
          idx_v, rows0, rows1, wtab_v, wvals, sem0, sem1, semw):
        wid = lax.axis_index("s") * 2 + lax.axis_index("c")
        base = wid * PERW
        pltpu.sync_copy(idx_hbm.at[pl.ds(base, PERW)], idx_v)
        pltpu.async_copy(wtab_hbm, wtab_v, semw)

        def ix(c):
            return idx_v.at[pl.ds(c * CHUNK, CHUNK)]

        def fire(c, buf, sem):
            pltpu.async_copy(etab_hbm.at[ix(c)], buf, sem)

        def drain(c, buf, sem):
            pltpu.make_async_copy(etab_hbm.at[ix(c)], buf, sem).wait()
            pltpu.sync_copy(buf, emb_out.at[pl.ds(base + c * CHUNK, CHUNK)])

        fire(0, rows0, sem0)
        fire(1, rows1, sem1)

        @pl.loop(0, NCH // 2 - 1)
        def _(g):
            c0 = g * 2
            drain(c0, rows0, sem0)
            fire(c0 + 2, rows0, sem0)
            drain(c0 + 1, rows1, sem1)
            fire(c0 + 3, rows1, sem1)

        drain(NCH - 2, rows0, sem0)
        drain(NCH - 1, rows1, sem1)

        pltpu.make_async_copy(wtab_hbm, wtab_v, semw).wait()

        @pl.loop(0, PERW // 16)
        def _(i):
            iv = idx_v[pl.ds(i * 16, 16)]
            wvals[pl.ds(i * 16, 16)] = plsc.load_gather(wtab_v, [iv])

        pltpu.sync_copy(wvals, w_out.at[pl.ds(base, PERW)])

    return k(idx_flat, e_table, w_flat)


def _fused_body(emb_ref, val_ref, wg_ref, w0_ref, w1_ref, bias_ref,
                r_ref, s_ref, wfca_ref, wfcb_ref, wfcc_ref, bfc_ref,
                out_ref, h0_s, h1_s, lin_s, fm_s, st0_s, st1_s):
    p = pl.program_id(0)
    i = pl.program_id(1)

    @pl.when(p == 0)
    def _():
        emb = emb_ref[...]
        val = val_ref[...]
        h0 = jnp.dot(emb, w0_ref[...], preferred_element_type=jnp.float32)
        h0 = h0 + bias_ref[0:1, :H]
        h0_s[pl.ds(i * BB, BB), :] = h0
        lin_s[pl.ds(i * BB, BB), :] = wg_ref[...] * val
        vexp = jnp.dot(val, r_ref[...], preferred_element_type=jnp.float32)
        t = emb * vexp
        s = jnp.dot(t, s_ref[...], preferred_element_type=jnp.float32)
        s2 = jnp.dot(t * t, s_ref[...], preferred_element_type=jnp.float32)
        fm_s[pl.ds(i * BB, BB), :] = 0.5 * (s * s - s2)
        ps = jnp.concatenate(
            [jnp.sum(h0, axis=0, keepdims=True),
             jnp.sum(h0 * h0, axis=0, keepdims=True)], axis=0)

        @pl.when(i == 0)
        def _():
            st0_s[...] = jnp.zeros_like(st0_s)

        st0_s[...] += ps

    @pl.when(p == 1)
    def _():
        st = st0_s[...]
        m = st[0:1, :] * (1.0 / B)
        v = st[1:2, :] * (1.0 / B) - m * m
        inv = bias_ref[1:2, :H] * lax.rsqrt(v + EPS)
        h0 = h0_s[pl.ds(i * BB, BB), :]
        a = jnp.maximum((h0 - m) * inv + bias_ref[2:3, :H], 0.0)
        h1 = jnp.dot(a, w1_ref[...], preferred_element_type=jnp.float32)
        h1 = h1 + bias_ref[3:4, :H]
        h1_s[pl.ds(i * BB, BB), :] = h1
        ps = jnp.concatenate(
            [jnp.sum(h1, axis=0, keepdims=True),
             jnp.sum(h1 * h1, axis=0, keepdims=True)], axis=0)

        @pl.when(i == 0)
        def _():
            st1_s[...] = jnp.zeros_like(st1_s)

        st1_s[...] += ps

    @pl.when(p == 2)
    def _():
        st = st1_s[...]
        m = st[0:1, :] * (1.0 / B)
        v = st[1:2, :] * (1.0 / B) - m * m
        inv = bias_ref[4:5, :H] * lax.rsqrt(v + EPS)
        h1 = h1_s[pl.ds(i * BB, BB), :]
        y = jnp.maximum((h1 - m) * inv + bias_ref[5:6, :H], 0.0)
        logit = (jnp.sum(lin_s[pl.ds(i * BB, BB), :] * wfca_ref[...],
                         axis=1, keepdims=True)
                 + jnp.sum(fm_s[pl.ds(i * BB, BB), :] * wfcb_ref[...],
                           axis=1, keepdims=True)
                 + jnp.dot(y, wfcc_ref[...],
                           preferred_element_type=jnp.float32)
                 + bfc_ref[...])
        out_ref[...] = jax.nn.sigmoid(logit)


def _tc_fused(emb, val, wg, W0, W1, bias6, R, S, wfca, wfcb, wfcc, bfc):
    def eb(p, i):
        return (jnp.where(p == 0, i, NB - 1), 0)

    def cst(p, i):
        return (0, 0)

    return pl.pallas_call(
        _fused_body,
        grid=(3, NB),
        in_specs=[
            pl.BlockSpec((BB, D_IN), eb),
            pl.BlockSpec((BB, F), eb),
            pl.BlockSpec((BB, F), eb),
            pl.BlockSpec((D_IN, H), cst),
            pl.BlockSpec((H, H), cst),
            pl.BlockSpec((6, H), cst),
            pl.BlockSpec((F, D_IN), cst),
            pl.BlockSpec((D_IN, E), cst),
            pl.BlockSpec((1, F), cst),
            pl.BlockSpec((1, E), cst),
            pl.BlockSpec((H, 1), cst),
            pl.BlockSpec((1, 1), cst),
        ],
        out_specs=pl.BlockSpec((BB, 1), lambda p, i: (jnp.where(p == 2, i, 0), 0)),
        out_shape=jax.ShapeDtypeStruct((B, 1), jnp.float32),
        scratch_shapes=[
            pltpu.VMEM((B, H), jnp.float32),
            pltpu.VMEM((B, H), jnp.float32),
            pltpu.VMEM((B, F), jnp.float32),
            pltpu.VMEM((B, E), jnp.float32),
            pltpu.VMEM((2, H), jnp.float32),
            pltpu.VMEM((2, H), jnp.float32),
        ],
    )(emb, val, wg, W0, W1, bias6, R, S, wfca, wfcb, wfcc, bfc)


def kernel(feat_idx, feat_val, w_table, e_table,
           W0, b0, g0, bt0, W1, b1, g1, bt1, Wfc, bfc):
    idx_flat = feat_idx.astype(jnp.int32).reshape(BF)
    emb_flat, wg_flat = _sc_gather(idx_flat, e_table, w_table.reshape(V))
    emb = emb_flat.reshape(B, D_IN)
    wg = wg_flat.reshape(B, F)
    val = feat_val.astype(jnp.float32)

    bias6 = jnp.stack([b0, g0, bt0, b1, g1, bt1], axis=0)
    R = jnp.repeat(jnp.eye(F, dtype=jnp.float32), E, axis=1)
    S = jnp.tile(jnp.eye(E, dtype=jnp.float32), (F, 1))

    return _tc_fused(emb, val, wg, W0, W1, bias6, R, S,
                     Wfc[:F].reshape(1, F), Wfc[F:F + E].reshape(1, E),
                     Wfc[F + E:], bfc.reshape(1, 1))


# speedup vs baseline: 1.9018x; 1.0149x over previous
import functools

import jax
import jax.numpy as jnp
from jax import lax
from jax.experimental import pallas as pl
from jax.experimental.pallas import tpu as pltpu
from jax.experimental.pallas import tpu_sc as plsc

B, F, V, E = 4096, 26, 100000, 64
D_IN = F * E
H = 400
BF = B * F
NW = 32
PERW = BF // NW
CHUNK = 128
NCH = PERW // CHUNK
BB = 512
NB = B // BB
EPS = 1e-3


def _sc_gather(idx_flat, e_table, w_flat):
    mesh = plsc.VectorSubcoreMesh(core_axis_name="c", subcore_axis_name="s")

    @functools.partial(
        pl.kernel,
        out_type=[
            jax.ShapeDtypeStruct((BF, E), jnp.float32),
            jax.ShapeDtypeStruct((BF,), jnp.float32),
        ],
        mesh=mesh,
        compiler_params=pltpu.CompilerParams(use_tc_tiling_on_sc=False,
                                             needs_layout_passes=False),
        scratch_types=[
            pltpu.VMEM((PERW,), jnp.int32),
            pltpu.VMEM((CHUNK, E), jnp.float32),
            pltpu.VMEM((CHUNK, E), jnp.float32),
            pltpu.VMEM((V,), jnp.float32),
            pltpu.VMEM((PERW,), jnp.float32),
            pltpu.SemaphoreType.DMA,
            pltpu.SemaphoreType.DMA,
            pltpu.SemaphoreType.DMA,
        ],
    )
    def k(idx_hbm, etab_flat_hbm, wtab_hbm, emb_out, w_out,
          idx_v, rows0, rows1, wtab_v, wvals, sem0, sem1, semw):
        etab_hbm = etab_flat_hbm
        wid = lax.axis_index("s") * 2 + lax.axis_index("c")
        base = wid * PERW
        pltpu.sync_copy(idx_hbm.at[pl.ds(base, PERW)], idx_v)
        pltpu.async_copy(wtab_hbm, wtab_v, semw)

        def ix(c):
            return idx_v.at[pl.ds(c * CHUNK, CHUNK)]

        def fire(c, buf, sem):
            pltpu.async_copy(etab_hbm.at[ix(c)], buf, sem)

        def drain(c, buf, sem):
            pltpu.make_async_copy(etab_hbm.at[ix(c)], buf, sem).wait()
            pltpu.sync_copy(buf, emb_out.at[pl.ds(base + c * CHUNK, CHUNK)])

        fire(0, rows0, sem0)
        fire(1, rows1, sem1)

        @pl.loop(0, NCH // 2 - 1)
        def _(g):
            c0 = g * 2
            drain(c0, rows0, sem0)
            fire(c0 + 2, rows0, sem0)
            drain(c0 + 1, rows1, sem1)
            fire(c0 + 3, rows1, sem1)

        drain(NCH - 2, rows0, sem0)
        drain(NCH - 1, rows1, sem1)

        pltpu.make_async_copy(wtab_hbm, wtab_v, semw).wait()

        @pl.loop(0, PERW // 16)
        def _(i):
            iv = idx_v[pl.ds(i * 16, 16)]
            wvals[pl.ds(i * 16, 16)] = plsc.load_gather(wtab_v, [iv])

        pltpu.sync_copy(wvals, w_out.at[pl.ds(base, PERW)])

    return k(idx_flat, e_table, w_flat)


def _fused_body(emb_ref, val_ref, wg_ref, w0_ref, w1_ref, bias_ref,
                r_ref, s_ref, wfca_ref, wfcb_ref, wfcc_ref, bfc_ref,
                out_ref, h0_s, h1_s, lin_s, fm_s, st0_s, st1_s):
    p = pl.program_id(0)
    i = pl.program_id(1)

    @pl.when(p == 0)
    def _():
        emb = emb_ref[...]
        val = val_ref[...]
        h0 = jnp.dot(emb.astype(jnp.bfloat16), w0_ref[...],
                     preferred_element_type=jnp.float32)
        h0 = h0 + bias_ref[0:1, :H]
        h0_s[pl.ds(i * BB, BB), :] = h0
        lin_s[pl.ds(i * BB, BB), :] = wg_ref[...] * val
        vexp = jnp.dot(val, r_ref[...], preferred_element_type=jnp.float32)
        t = emb * vexp
        s = jnp.dot(t.astype(jnp.bfloat16), s_ref[...],
                    preferred_element_type=jnp.float32)
        s2 = jnp.dot((t * t).astype(jnp.bfloat16), s_ref[...],
                     preferred_element_type=jnp.float32)
        fm_s[pl.ds(i * BB, BB), :] = 0.5 * (s * s - s2)
        ps = jnp.concatenate(
            [jnp.sum(h0, axis=0, keepdims=True),
             jnp.sum(h0 * h0, axis=0, keepdims=True)], axis=0)

        @pl.when(i == 0)
        def _():
            st0_s[...] = jnp.zeros_like(st0_s)

        st0_s[...] += ps

    @pl.when(p == 1)
    def _():
        st = st0_s[...]
        m = st[0:1, :] * (1.0 / B)
        v = st[1:2, :] * (1.0 / B) - m * m
        inv = bias_ref[1:2, :H] * lax.rsqrt(v + EPS)
        h0 = h0_s[pl.ds(i * BB, BB), :]
        a = jnp.maximum((h0 - m) * inv + bias_ref[2:3, :H], 0.0)
        h1 = jnp.dot(a.astype(jnp.bfloat16), w1_ref[...],
                     preferred_element_type=jnp.float32)
        h1 = h1 + bias_ref[3:4, :H]
        h1_s[pl.ds(i * BB, BB), :] = h1
        ps = jnp.concatenate(
            [jnp.sum(h1, axis=0, keepdims=True),
             jnp.sum(h1 * h1, axis=0, keepdims=True)], axis=0)

        @pl.when(i == 0)
        def _():
            st1_s[...] = jnp.zeros_like(st1_s)

        st1_s[...] += ps

    @pl.when(p == 2)
    def _():
        st = st1_s[...]
        m = st[0:1, :] * (1.0 / B)
        v = st[1:2, :] * (1.0 / B) - m * m
        inv = bias_ref[4:5, :H] * lax.rsqrt(v + EPS)
        h1 = h1_s[pl.ds(i * BB, BB), :]
        y = jnp.maximum((h1 - m) * inv + bias_ref[5:6, :H], 0.0)
        logit = (jnp.sum(lin_s[pl.ds(i * BB, BB), :] * wfca_ref[...],
                         axis=1, keepdims=True)
                 + jnp.sum(fm_s[pl.ds(i * BB, BB), :] * wfcb_ref[...],
                           axis=1, keepdims=True)
                 + jnp.dot(y, wfcc_ref[...],
                           preferred_element_type=jnp.float32)
                 + bfc_ref[...])
        out_ref[...] = jax.nn.sigmoid(logit)


def _tc_fused(emb, val, wg, W0, W1, bias6, R, S, wfca, wfcb, wfcc, bfc):
    def eb(p, i):
        return (jnp.where(p == 0, i, NB - 1), 0)

    def cst(p, i):
        return (0, 0)

    return pl.pallas_call(
        _fused_body,
        grid=(3, NB),
        in_specs=[
            pl.BlockSpec((BB, D_IN), eb),
            pl.BlockSpec((BB, F), eb),
            pl.BlockSpec((BB, F), eb),
            pl.BlockSpec((D_IN, H), cst),
            pl.BlockSpec((H, H), cst),
            pl.BlockSpec((6, H), cst),
            pl.BlockSpec((F, D_IN), cst),
            pl.BlockSpec((D_IN, E), cst),
            pl.BlockSpec((1, F), cst),
            pl.BlockSpec((1, E), cst),
            pl.BlockSpec((H, 1), cst),
            pl.BlockSpec((1, 1), cst),
        ],
        out_specs=pl.BlockSpec((BB, 1), lambda p, i: (jnp.where(p == 2, i, 0), 0)),
        out_shape=jax.ShapeDtypeStruct((B, 1), jnp.float32),
        scratch_shapes=[
            pltpu.VMEM((B, H), jnp.float32),
            pltpu.VMEM((B, H), jnp.float32),
            pltpu.VMEM((B, F), jnp.float32),
            pltpu.VMEM((B, E), jnp.float32),
            pltpu.VMEM((2, H), jnp.float32),
            pltpu.VMEM((2, H), jnp.float32),
        ],
    )(emb, val, wg, W0, W1, bias6, R, S, wfca, wfcb, wfcc, bfc)


def kernel(feat_idx, feat_val, w_table, e_table,
           W0, b0, g0, bt0, W1, b1, g1, bt1, Wfc, bfc):
    idx_flat = feat_idx.astype(jnp.int32).reshape(BF)
    emb_flat, wg_flat = _sc_gather(idx_flat, e_table, w_table.reshape(V))
    emb = emb_flat.reshape(B, D_IN)
    wg = wg_flat.reshape(B, F)
    val = feat_val.astype(jnp.float32)

    bias6 = jnp.stack([b0, g0, bt0, b1, g1, bt1], axis=0)
    R = jnp.repeat(jnp.eye(F, dtype=jnp.float32), E, axis=1)
    S = jnp.tile(jnp.eye(E, dtype=jnp.bfloat16), (F, 1))

    return _tc_fused(emb, val, wg, W0.astype(jnp.bfloat16),
                     W1.astype(jnp.bfloat16), bias6, R, S,
                     Wfc[:F].reshape(1, F), Wfc[F:F + E].reshape(1, E),
                     Wfc[F + E:], bfc.reshape(1, 1))


# speedup vs baseline: 2.0651x; 1.0859x over previous
import functools

import jax
import jax.numpy as jnp
from jax import lax
from jax.experimental import pallas as pl
from jax.experimental.pallas import tpu as pltpu
from jax.experimental.pallas import tpu_sc as plsc

B, F, V, E = 4096, 26, 100000, 64
D_IN = F * E
H = 400
BF = B * F
NW = 32
PERW = BF // NW
CHUNK = 128
NCH = PERW // CHUNK
BB = 512
NB = B // BB
EPS = 1e-3


NS = D_IN // 128


def _sc_gather(idx_flat, dest3, e_table, w_flat):
    mesh = plsc.VectorSubcoreMesh(core_axis_name="c", subcore_axis_name="s")

    @functools.partial(
        pl.kernel,
        out_type=[
            jax.ShapeDtypeStruct((BF, E), jnp.float32),
            jax.ShapeDtypeStruct((BF,), jnp.float32),
        ],
        mesh=mesh,
        compiler_params=pltpu.CompilerParams(use_tc_tiling_on_sc=False,
                                             needs_layout_passes=False),
        scratch_types=[
            pltpu.VMEM((PERW,), jnp.int32),
            pltpu.VMEM((NCH, CHUNK), jnp.int32),
            pltpu.VMEM((CHUNK, E), jnp.float32),
            pltpu.VMEM((CHUNK, E), jnp.float32),
            pltpu.VMEM((V,), jnp.float32),
            pltpu.VMEM((PERW,), jnp.float32),
            pltpu.SemaphoreType.DMA,
            pltpu.SemaphoreType.DMA,
            pltpu.SemaphoreType.DMA,
            pltpu.SemaphoreType.DMA,
        ],
    )
    def k(idx_hbm, dest_hbm, etab_hbm, wtab_hbm, emb_out, w_out,
          idx_v, dest_v, rows0, rows1, wtab_v, wvals, sem0, sem1, sems, semw):
        wid = lax.axis_index("s") * 2 + lax.axis_index("c")
        base = wid * PERW
        pltpu.sync_copy(idx_hbm.at[pl.ds(base, PERW)], idx_v)
        pltpu.sync_copy(dest_hbm.at[wid], dest_v)
        pltpu.async_copy(wtab_hbm, wtab_v, semw)

        def ix(c):
            return idx_v.at[pl.ds(c * CHUNK, CHUNK)]

        def fire(c, buf, sem):
            pltpu.async_copy(etab_hbm.at[ix(c)], buf, sem)

        def drain(c, buf, sem):
            pltpu.make_async_copy(etab_hbm.at[ix(c)], buf, sem).wait()
            pltpu.async_copy(buf, emb_out.at[dest_v.at[c]], sems).wait()

        fire(0, rows0, sem0)
        fire(1, rows1, sem1)

        @pl.loop(0, NCH // 2 - 1)
        def _(g):
            c0 = g * 2
            drain(c0, rows0, sem0)
            fire(c0 + 2, rows0, sem0)
            drain(c0 + 1, rows1, sem1)
            fire(c0 + 3, rows1, sem1)

        drain(NCH - 2, rows0, sem0)
        drain(NCH - 1, rows1, sem1)

        pltpu.make_async_copy(wtab_hbm, wtab_v, semw).wait()

        @pl.loop(0, PERW // 16)
        def _(i):
            iv = idx_v[pl.ds(i * 16, 16)]
            wvals[pl.ds(i * 16, 16)] = plsc.load_gather(wtab_v, [iv])

        pltpu.sync_copy(wvals, w_out.at[pl.ds(base, PERW)])

    return k(idx_flat, dest3, e_table, w_flat)


def _fused_body(*refs):
    (e_refs, (val_ref, wg_ref, w0_ref, w1_ref, bias_ref,
              r_ref, s_ref, wfca_ref, wfcb_ref, wfcc_ref, bfc_ref,
              out_ref, h0_s, h1_s, lin_s, fm_s, st0_s, st1_s)) = \
        refs[:NS], refs[NS:]
    p = pl.program_id(0)
    i = pl.program_id(1)

    @pl.when(p == 0)
    def _():
        val = val_ref[...]
        vexp = jnp.dot(val, r_ref[...], preferred_element_type=jnp.float32)
        h0 = None
        s = None
        s2 = None
        for c in range(NS):
            ec = e_refs[c][...]
            pc = jnp.dot(ec.astype(jnp.bfloat16),
                         w0_ref[pl.ds(c * 128, 128), :],
                         preferred_element_type=jnp.float32)
            t = ec * vexp[:, c * 128:(c + 1) * 128]
            sc_ = jnp.dot(t.astype(jnp.bfloat16),
                          s_ref[pl.ds(c * 128, 128), :],
                          preferred_element_type=jnp.float32)
            s2c = jnp.dot((t * t).astype(jnp.bfloat16),
                          s_ref[pl.ds(c * 128, 128), :],
                          preferred_element_type=jnp.float32)
            h0 = pc if h0 is None else h0 + pc
            s = sc_ if s is None else s + sc_
            s2 = s2c if s2 is None else s2 + s2c
        h0 = h0 + bias_ref[0:1, :H]
        h0_s[pl.ds(i * BB, BB), :] = h0
        lin_s[pl.ds(i * BB, BB), :] = wg_ref[...] * val
        fm_s[pl.ds(i * BB, BB), :] = 0.5 * (s * s - s2)
        ps = jnp.concatenate(
            [jnp.sum(h0, axis=0, keepdims=True),
             jnp.sum(h0 * h0, axis=0, keepdims=True)], axis=0)

        @pl.when(i == 0)
        def _():
            st0_s[...] = jnp.zeros_like(st0_s)

        st0_s[...] += ps

    @pl.when(p == 1)
    def _():
        st = st0_s[...]
        m = st[0:1, :] * (1.0 / B)
        v = st[1:2, :] * (1.0 / B) - m * m
        inv = bias_ref[1:2, :H] * lax.rsqrt(v + EPS)
        h0 = h0_s[pl.ds(i * BB, BB), :]
        a = jnp.maximum((h0 - m) * inv + bias_ref[2:3, :H], 0.0)
        h1 = jnp.dot(a.astype(jnp.bfloat16), w1_ref[...],
                     preferred_element_type=jnp.float32)
        h1 = h1 + bias_ref[3:4, :H]
        h1_s[pl.ds(i * BB, BB), :] = h1
        ps = jnp.concatenate(
            [jnp.sum(h1, axis=0, keepdims=True),
             jnp.sum(h1 * h1, axis=0, keepdims=True)], axis=0)

        @pl.when(i == 0)
        def _():
            st1_s[...] = jnp.zeros_like(st1_s)

        st1_s[...] += ps

    @pl.when(p == 2)
    def _():
        st = st1_s[...]
        m = st[0:1, :] * (1.0 / B)
        v = st[1:2, :] * (1.0 / B) - m * m
        inv = bias_ref[4:5, :H] * lax.rsqrt(v + EPS)
        h1 = h1_s[pl.ds(i * BB, BB), :]
        y = jnp.maximum((h1 - m) * inv + bias_ref[5:6, :H], 0.0)
        logit = (jnp.sum(lin_s[pl.ds(i * BB, BB), :] * wfca_ref[...],
                         axis=1, keepdims=True)
                 + jnp.sum(fm_s[pl.ds(i * BB, BB), :] * wfcb_ref[...],
                           axis=1, keepdims=True)
                 + jnp.dot(y, wfcc_ref[...],
                           preferred_element_type=jnp.float32)
                 + bfc_ref[...])
        out_ref[...] = jax.nn.sigmoid(logit)


def _tc_fused(emb2, val, wg, W0, W1, bias6, R, S, wfca, wfcb, wfcc, bfc):
    def eb(p, i):
        return (jnp.where(p == 0, i, NB - 1), 0)

    def slab(c):
        return pl.BlockSpec(
            (BB, 128), lambda p, i, c=c: (jnp.where(p == 0, i, NB - 1) + c * NB, 0))

    def cst(p, i):
        return (0, 0)

    return pl.pallas_call(
        _fused_body,
        grid=(3, NB),
        in_specs=[slab(c) for c in range(NS)] + [
            pl.BlockSpec((BB, F), eb),
            pl.BlockSpec((BB, F), eb),
            pl.BlockSpec((D_IN, H), cst),
            pl.BlockSpec((H, H), cst),
            pl.BlockSpec((6, H), cst),
            pl.BlockSpec((F, D_IN), cst),
            pl.BlockSpec((D_IN, E), cst),
            pl.BlockSpec((1, F), cst),
            pl.BlockSpec((1, E), cst),
            pl.BlockSpec((H, 1), cst),
            pl.BlockSpec((1, 1), cst),
        ],
        out_specs=pl.BlockSpec((BB, 1), lambda p, i: (jnp.where(p == 2, i, 0), 0)),
        out_shape=jax.ShapeDtypeStruct((B, 1), jnp.float32),
        scratch_shapes=[
            pltpu.VMEM((B, H), jnp.float32),
            pltpu.VMEM((B, H), jnp.float32),
            pltpu.VMEM((B, F), jnp.float32),
            pltpu.VMEM((B, E), jnp.float32),
            pltpu.VMEM((2, H), jnp.float32),
            pltpu.VMEM((2, H), jnp.float32),
        ],
    )(*([emb2] * NS), val, wg, W0, W1, bias6, R, S, wfca, wfcb, wfcc, bfc)


def kernel(feat_idx, feat_val, w_table, e_table,
           W0, b0, g0, bt0, W1, b1, g1, bt1, Wfc, bfc):
    idx_flat = feat_idx.astype(jnp.int32).reshape(BF)
    j = jnp.arange(BF, dtype=jnp.int32)
    b = j // F
    f = j % F
    dest3 = ((f >> 1) * (B * 2) + (b >> 3) * 16 + (b & 7) * 2
             + (f & 1)).reshape(NW, NCH, CHUNK)
    emb_flat, wg_flat = _sc_gather(idx_flat, dest3, e_table,
                                   w_table.reshape(V))
    emb2 = emb_flat.reshape(BF // 2, 2 * E)
    wg = wg_flat.reshape(B, F)
    val = feat_val.astype(jnp.float32)

    bias6 = jnp.stack([b0, g0, bt0, b1, g1, bt1], axis=0)
    R = jnp.repeat(jnp.eye(F, dtype=jnp.float32), E, axis=1)
    S = jnp.tile(jnp.eye(E, dtype=jnp.bfloat16), (F, 1))

    return _tc_fused(emb2, val, wg, W0.astype(jnp.bfloat16),
                     W1.astype(jnp.bfloat16), bias6, R, S,
                     Wfc[:F].reshape(1, F), Wfc[F:F + E].reshape(1, E),
                     Wfc[F + E:], bfc.reshape(1, 1))


# speedup vs baseline: 2.2528x; 1.0909x over previous
import functools

import jax
import jax.numpy as jnp
from jax import lax
from jax.experimental import pallas as pl
from jax.experimental.pallas import tpu as pltpu
from jax.experimental.pallas import tpu_sc as plsc

B, F, V, E = 4096, 26, 100000, 64
D_IN = F * E
H = 400
BF = B * F
NW = 32
PERW = BF // NW
CHUNK = 128
NCH = PERW // CHUNK
BB = 512
NB = B // BB
EPS = 1e-3


NS = D_IN // 128


def _sc_gather(idx_flat, dest3, e_table, w_flat):
    mesh = plsc.VectorSubcoreMesh(core_axis_name="c", subcore_axis_name="s")

    @functools.partial(
        pl.kernel,
        out_type=[
            jax.ShapeDtypeStruct((BF, E), jnp.float32),
            jax.ShapeDtypeStruct((BF,), jnp.float32),
        ],
        mesh=mesh,
        compiler_params=pltpu.CompilerParams(use_tc_tiling_on_sc=False,
                                             needs_layout_passes=False),
        scratch_types=[
            pltpu.VMEM((PERW,), jnp.int32),
            pltpu.VMEM((NCH, CHUNK), jnp.int32),
            pltpu.VMEM((CHUNK, E), jnp.float32),
            pltpu.VMEM((CHUNK, E), jnp.float32),
            pltpu.VMEM((PERW,), jnp.float32),
            pltpu.SemaphoreType.DMA,
            pltpu.SemaphoreType.DMA,
            pltpu.SemaphoreType.DMA,
            pltpu.SemaphoreType.DMA,
            pltpu.SemaphoreType.DMA,
        ],
    )
    def k(idx_hbm, dest_hbm, etab_hbm, wtab_hbm, emb_out, w_out,
          idx_v, dest_v, rows0, rows1, wvals, sem0, sem1, ss0, ss1, semw):
        wid = lax.axis_index("s") * 2 + lax.axis_index("c")
        base = wid * PERW
        pltpu.sync_copy(idx_hbm.at[pl.ds(base, PERW)], idx_v)
        pltpu.sync_copy(dest_hbm.at[wid], dest_v)

        def ix(c):
            return idx_v.at[pl.ds(c * CHUNK, CHUNK)]

        def gfire(c, buf, sem):
            pltpu.async_copy(etab_hbm.at[ix(c)], buf, sem)

        def gwait(c, buf, sem):
            pltpu.make_async_copy(etab_hbm.at[ix(c)], buf, sem).wait()

        def sfire(c, buf, sem):
            pltpu.async_copy(buf, emb_out.at[dest_v.at[c]], sem)

        def swait(c, buf, sem):
            pltpu.make_async_copy(buf, emb_out.at[dest_v.at[c]], sem).wait()

        def wfire(c):
            pltpu.async_copy(wtab_hbm.at[ix(c)],
                             wvals.at[pl.ds(c * CHUNK, CHUNK)], semw)

        gfire(0, rows0, sem0)
        wfire(0)
        gfire(1, rows1, sem1)
        wfire(1)
        gwait(0, rows0, sem0)
        sfire(0, rows0, ss0)
        gwait(1, rows1, sem1)
        sfire(1, rows1, ss1)

        @pl.loop(1, NCH // 2)
        def _(g):
            c0 = 2 * g
            c1 = c0 + 1
            swait(c0 - 2, rows0, ss0)
            gfire(c0, rows0, sem0)
            wfire(c0)
            swait(c1 - 2, rows1, ss1)
            gfire(c1, rows1, sem1)
            wfire(c1)
            gwait(c0, rows0, sem0)
            sfire(c0, rows0, ss0)
            gwait(c1, rows1, sem1)
            sfire(c1, rows1, ss1)

        swait(NCH - 2, rows0, ss0)
        swait(NCH - 1, rows1, ss1)

        @pl.loop(0, NCH)
        def _(c):
            pltpu.make_async_copy(wtab_hbm.at[ix(c)],
                                  wvals.at[pl.ds(c * CHUNK, CHUNK)],
                                  semw).wait()

        pltpu.sync_copy(wvals, w_out.at[pl.ds(base, PERW)])

    return k(idx_flat, dest3, e_table, w_flat)


def _fused_body(*refs):
    (e_refs, (val_ref, wg_ref, w0_ref, w1_ref, bias_ref,
              r_ref, wfca_ref, wfcb_ref, wfcc_ref, bfc_ref,
              out_ref, h0_s, h1_s, lin_s, fm_s, st0_s, st1_s)) = \
        refs[:NS], refs[NS:]
    p = pl.program_id(0)
    i = pl.program_id(1)

    @pl.when(p == 0)
    def _():
        val = val_ref[...]
        vexp = jnp.dot(val, r_ref[...], preferred_element_type=jnp.float32)
        h0 = None
        s = None
        s2 = None
        for c in range(NS):
            ec = e_refs[c][...]
            pc = jnp.dot(ec.astype(jnp.bfloat16),
                         w0_ref[pl.ds(c * 128, 128), :],
                         preferred_element_type=jnp.float32)
            t = ec * vexp[:, c * 128:(c + 1) * 128]
            tsq = t * t
            sc_ = t[:, :E] + t[:, E:]
            s2c = tsq[:, :E] + tsq[:, E:]
            h0 = pc if h0 is None else h0 + pc
            s = sc_ if s is None else s + sc_
            s2 = s2c if s2 is None else s2 + s2c
        h0 = h0 + bias_ref[0:1, :H]
        h0_s[pl.ds(i * BB, BB), :] = h0
        lin_s[pl.ds(i * BB, BB), :] = wg_ref[...] * val
        fm_s[pl.ds(i * BB, BB), :] = 0.5 * (s * s - s2)
        ps = jnp.concatenate(
            [jnp.sum(h0, axis=0, keepdims=True),
             jnp.sum(h0 * h0, axis=0, keepdims=True)], axis=0)

        @pl.when(i == 0)
        def _():
            st0_s[...] = jnp.zeros_like(st0_s)

        st0_s[...] += ps

    @pl.when(p == 1)
    def _():
        st = st0_s[...]
        m = st[0:1, :] * (1.0 / B)
        v = st[1:2, :] * (1.0 / B) - m * m
        inv = bias_ref[1:2, :H] * lax.rsqrt(v + EPS)
        h0 = h0_s[pl.ds(i * BB, BB), :]
        a = jnp.maximum((h0 - m) * inv + bias_ref[2:3, :H], 0.0)
        h1 = jnp.dot(a.astype(jnp.bfloat16), w1_ref[...],
                     preferred_element_type=jnp.float32)
        h1 = h1 + bias_ref[3:4, :H]
        h1_s[pl.ds(i * BB, BB), :] = h1
        ps = jnp.concatenate(
            [jnp.sum(h1, axis=0, keepdims=True),
             jnp.sum(h1 * h1, axis=0, keepdims=True)], axis=0)

        @pl.when(i == 0)
        def _():
            st1_s[...] = jnp.zeros_like(st1_s)

        st1_s[...] += ps

    @pl.when(p == 2)
    def _():
        st = st1_s[...]
        m = st[0:1, :] * (1.0 / B)
        v = st[1:2, :] * (1.0 / B) - m * m
        inv = bias_ref[4:5, :H] * lax.rsqrt(v + EPS)
        h1 = h1_s[pl.ds(i * BB, BB), :]
        y = jnp.maximum((h1 - m) * inv + bias_ref[5:6, :H], 0.0)
        logit = (jnp.sum(lin_s[pl.ds(i * BB, BB), :] * wfca_ref[...],
                         axis=1, keepdims=True)
                 + jnp.sum(fm_s[pl.ds(i * BB, BB), :] * wfcb_ref[...],
                           axis=1, keepdims=True)
                 + jnp.dot(y, wfcc_ref[...],
                           preferred_element_type=jnp.float32)
                 + bfc_ref[...])
        out_ref[...] = jax.nn.sigmoid(logit)


def _tc_fused(emb2, val, wg, W0, W1, bias6, R, wfca, wfcb, wfcc, bfc):
    def eb(p, i):
        return (jnp.where(p == 0, i, NB - 1), 0)

    def slab(c):
        return pl.BlockSpec(
            (BB, 128), lambda p, i, c=c: (jnp.where(p == 0, i, NB - 1) + c * NB, 0))

    def cst(p, i):
        return (0, 0)

    return pl.pallas_call(
        _fused_body,
        grid=(3, NB),
        in_specs=[slab(c) for c in range(NS)] + [
            pl.BlockSpec((BB, F), eb),
            pl.BlockSpec((BB, F), eb),
            pl.BlockSpec((D_IN, H), cst),
            pl.BlockSpec((H, H), cst),
            pl.BlockSpec((6, H), cst),
            pl.BlockSpec((F, D_IN), cst),
            pl.BlockSpec((1, F), cst),
            pl.BlockSpec((1, E), cst),
            pl.BlockSpec((H, 1), cst),
            pl.BlockSpec((1, 1), cst),
        ],
        out_specs=pl.BlockSpec((BB, 1), lambda p, i: (jnp.where(p == 2, i, 0), 0)),
        out_shape=jax.ShapeDtypeStruct((B, 1), jnp.float32),
        scratch_shapes=[
            pltpu.VMEM((B, H), jnp.float32),
            pltpu.VMEM((B, H), jnp.float32),
            pltpu.VMEM((B, F), jnp.float32),
            pltpu.VMEM((B, E), jnp.float32),
            pltpu.VMEM((2, H), jnp.float32),
            pltpu.VMEM((2, H), jnp.float32),
        ],
    )(*([emb2] * NS), val, wg, W0, W1, bias6, R, wfca, wfcb, wfcc, bfc)


def kernel(feat_idx, feat_val, w_table, e_table,
           W0, b0, g0, bt0, W1, b1, g1, bt1, Wfc, bfc):
    idx_flat = feat_idx.astype(jnp.int32).reshape(BF)
    j = jnp.arange(BF, dtype=jnp.int32)
    b = j // F
    f = j % F
    dest3 = ((f >> 1) * (B * 2) + (b >> 3) * 16 + (b & 7) * 2
             + (f & 1)).reshape(NW, NCH, CHUNK)
    emb_flat, wg_flat = _sc_gather(idx_flat, dest3, e_table,
                                   w_table.reshape(V))
    emb2 = emb_flat.reshape(BF // 2, 2 * E)
    wg = wg_flat.reshape(B, F)
    val = feat_val.astype(jnp.float32)

    bias6 = jnp.stack([b0, g0, bt0, b1, g1, bt1], axis=0)
    R = jnp.repeat(jnp.eye(F, dtype=jnp.float32), E, axis=1)

    return _tc_fused(emb2, val, wg, W0.astype(jnp.bfloat16),
                     W1.astype(jnp.bfloat16), bias6, R,
                     Wfc[:F].reshape(1, F), Wfc[F:F + E].reshape(1, E),
                     Wfc[F + E:], bfc.reshape(1, 1))


# speedup vs baseline: 2.2532x; 1.0002x over previous
import functools

import jax
import jax.numpy as jnp
from jax import lax
from jax.experimental import pallas as pl
from jax.experimental.pallas import tpu as pltpu
from jax.experimental.pallas import tpu_sc as plsc

B, F, V, E = 4096, 26, 100000, 64
D_IN = F * E
H = 400
BF = B * F
NW = 32
PERW = BF // NW
CHUNK = 128
NCH = PERW // CHUNK
BB = 512
NB = B // BB
EPS = 1e-3


NS = D_IN // 128


def _sc_gather(idx_flat, dest3, e_table, w_flat):
    mesh = plsc.VectorSubcoreMesh(core_axis_name="c", subcore_axis_name="s")

    @functools.partial(
        pl.kernel,
        out_type=[
            jax.ShapeDtypeStruct((BF, E), jnp.float32),
            jax.ShapeDtypeStruct((BF,), jnp.float32),
        ],
        mesh=mesh,
        compiler_params=pltpu.CompilerParams(use_tc_tiling_on_sc=False,
                                             needs_layout_passes=False),
        scratch_types=[
            pltpu.VMEM((PERW,), jnp.int32),
            pltpu.VMEM((NCH, CHUNK), jnp.int32),
            pltpu.VMEM((CHUNK, E), jnp.float32),
            pltpu.VMEM((CHUNK, E), jnp.float32),
            pltpu.VMEM((PERW,), jnp.float32),
            pltpu.SemaphoreType.DMA,
            pltpu.SemaphoreType.DMA,
            pltpu.SemaphoreType.DMA,
            pltpu.SemaphoreType.DMA,
            pltpu.SemaphoreType.DMA,
        ],
    )
    def k(idx_hbm, dest_hbm, etab_hbm, wtab_hbm, emb_out, w_out,
          idx_v, dest_v, rows0, rows1, wvals, sem0, sem1, ss0, ss1, semw):
        wid = lax.axis_index("s") * 2 + lax.axis_index("c")
        base = wid * PERW
        pltpu.sync_copy(idx_hbm.at[pl.ds(base, PERW)], idx_v)
        pltpu.sync_copy(dest_hbm.at[wid], dest_v)

        def ix(c):
            return idx_v.at[pl.ds(c * CHUNK, CHUNK)]

        def gfire(c, buf, sem):
            pltpu.async_copy(etab_hbm.at[ix(c)], buf, sem)

        def gwait(c, buf, sem):
            pltpu.make_async_copy(etab_hbm.at[ix(c)], buf, sem).wait()

        def sfire(c, buf, sem):
            pltpu.async_copy(buf, emb_out.at[dest_v.at[c]], sem)

        def swait(c, buf, sem):
            pltpu.make_async_copy(buf, emb_out.at[dest_v.at[c]], sem).wait()

        def wfire(c):
            pltpu.async_copy(wtab_hbm.at[ix(c)],
                             wvals.at[pl.ds(c * CHUNK, CHUNK)], semw)

        gfire(0, rows0, sem0)
        wfire(0)
        gfire(1, rows1, sem1)
        wfire(1)
        gwait(0, rows0, sem0)
        sfire(0, rows0, ss0)
        gwait(1, rows1, sem1)
        sfire(1, rows1, ss1)

        @pl.loop(1, NCH // 2)
        def _(g):
            c0 = 2 * g
            c1 = c0 + 1
            swait(c0 - 2, rows0, ss0)
            gfire(c0, rows0, sem0)
            wfire(c0)
            swait(c1 - 2, rows1, ss1)
            gfire(c1, rows1, sem1)
            wfire(c1)
            gwait(c0, rows0, sem0)
            sfire(c0, rows0, ss0)
            gwait(c1, rows1, sem1)
            sfire(c1, rows1, ss1)

        swait(NCH - 2, rows0, ss0)
        swait(NCH - 1, rows1, ss1)

        @pl.loop(0, NCH)
        def _(c):
            pltpu.make_async_copy(wtab_hbm.at[ix(c)],
                                  wvals.at[pl.ds(c * CHUNK, CHUNK)],
                                  semw).wait()

        pltpu.sync_copy(wvals, w_out.at[pl.ds(base, PERW)])

    return k(idx_flat, dest3, e_table, w_flat)


def _fused_body(*refs):
    (e_refs, (val_ref, wg_ref, w0_ref, w1_ref, bias_ref,
              r_ref, wfca_ref, wfcb_ref, wfcc_ref, bfc_ref,
              out_ref, h0_s, h1_s, lin_s, fm_s, st0_s, st1_s)) = \
        refs[:NS], refs[NS:]
    p = pl.program_id(0)
    i = pl.program_id(1)

    @pl.when(p == 0)
    def _():
        val = val_ref[...]
        vexp = jnp.dot(val, r_ref[...], preferred_element_type=jnp.float32)
        h0 = None
        s = None
        s2 = None
        for c in range(NS):
            ec = e_refs[c][...]
            pc = jnp.dot(ec.astype(jnp.bfloat16),
                         w0_ref[pl.ds(c * 128, 128), :],
                         preferred_element_type=jnp.float32)
            t = ec * vexp[:, c * 128:(c + 1) * 128]
            tsq = t * t
            sc_ = t[:, :E] + t[:, E:]
            s2c = tsq[:, :E] + tsq[:, E:]
            h0 = pc if h0 is None else h0 + pc
            s = sc_ if s is None else s + sc_
            s2 = s2c if s2 is None else s2 + s2c
        h0 = h0 + bias_ref[0:1, :H]
        h0_s[pl.ds(i * BB, BB), :] = h0
        lin_s[pl.ds(i * BB, BB), :] = wg_ref[...] * val
        fm_s[pl.ds(i * BB, BB), :] = 0.5 * (s * s - s2)
        ps = jnp.concatenate(
            [jnp.sum(h0, axis=0, keepdims=True),
             jnp.sum(h0 * h0, axis=0, keepdims=True)], axis=0)

        @pl.when(i == 0)
        def _():
            st0_s[...] = jnp.zeros_like(st0_s)

        st0_s[...] += ps

    @pl.when(p == 1)
    def _():
        st = st0_s[...]
        m = st[0:1, :] * (1.0 / B)
        v = st[1:2, :] * (1.0 / B) - m * m
        inv = bias_ref[1:2, :H] * lax.rsqrt(v + EPS)
        h0 = h0_s[pl.ds(i * BB, BB), :]
        a = jnp.maximum((h0 - m) * inv + bias_ref[2:3, :H], 0.0)
        h1 = jnp.dot(a.astype(jnp.bfloat16), w1_ref[...],
                     preferred_element_type=jnp.float32)
        h1 = h1 + bias_ref[3:4, :H]
        h1_s[pl.ds(i * BB, BB), :] = h1
        ps = jnp.concatenate(
            [jnp.sum(h1, axis=0, keepdims=True),
             jnp.sum(h1 * h1, axis=0, keepdims=True)], axis=0)

        @pl.when(i == 0)
        def _():
            st1_s[...] = jnp.zeros_like(st1_s)

        st1_s[...] += ps

    @pl.when(p == 2)
    def _():
        st = st1_s[...]
        m = st[0:1, :] * (1.0 / B)
        v = st[1:2, :] * (1.0 / B) - m * m
        inv = bias_ref[4:5, :H] * lax.rsqrt(v + EPS)
        h1 = h1_s[pl.ds(i * BB, BB), :]
        y = jnp.maximum((h1 - m) * inv + bias_ref[5:6, :H], 0.0)
        logit = (jnp.sum(lin_s[pl.ds(i * BB, BB), :] * wfca_ref[...],
                         axis=1, keepdims=True)
                 + jnp.sum(fm_s[pl.ds(i * BB, BB), :] * wfcb_ref[...],
                           axis=1, keepdims=True)
                 + jnp.dot(y, wfcc_ref[...],
                           preferred_element_type=jnp.float32)
                 + bfc_ref[...])
        out_ref[...] = jax.nn.sigmoid(logit)


def _tc_fused(emb2, val, wg, W0, W1, bias6, R, wfca, wfcb, wfcc, bfc):
    def eb(p, i):
        return (jnp.where(p == 0, i, NB - 1), 0)

    def slab(c):
        return pl.BlockSpec(
            (BB, 128), lambda p, i, c=c: (jnp.where(p == 0, i, NB - 1) + c * NB, 0))

    def cst(p, i):
        return (0, 0)

    return pl.pallas_call(
        _fused_body,
        grid=(3, NB),
        in_specs=[slab(c) for c in range(NS)] + [
            pl.BlockSpec((BB, F), eb),
            pl.BlockSpec((BB, F), eb),
            pl.BlockSpec((D_IN, H), cst),
            pl.BlockSpec((H, H), cst),
            pl.BlockSpec((6, H), cst),
            pl.BlockSpec((F, D_IN), cst),
            pl.BlockSpec((1, F), cst),
            pl.BlockSpec((1, E), cst),
            pl.BlockSpec((H, 1), cst),
            pl.BlockSpec((1, 1), cst),
        ],
        out_specs=pl.BlockSpec((BB, 1), lambda p, i: (jnp.where(p == 2, i, 0), 0)),
        out_shape=jax.ShapeDtypeStruct((B, 1), jnp.float32),
        scratch_shapes=[
            pltpu.VMEM((B, H), jnp.float32),
            pltpu.VMEM((B, H), jnp.float32),
            pltpu.VMEM((B, F), jnp.float32),
            pltpu.VMEM((B, E), jnp.float32),
            pltpu.VMEM((2, H), jnp.float32),
            pltpu.VMEM((2, H), jnp.float32),
        ],
    )(*([emb2] * NS), val, wg, W0, W1, bias6, R, wfca, wfcb, wfcc, bfc)


def kernel(feat_idx, feat_val, w_table, e_table,
           W0, b0, g0, bt0, W1, b1, g1, bt1, Wfc, bfc):
    idx_flat = feat_idx.astype(jnp.int32).reshape(BF)
    j = jnp.arange(BF, dtype=jnp.int32)
    b = j // F
    f = j % F
    dest3 = ((f >> 1) * (B * 2) + (b >> 3) * 16 + (b & 7) * 2
             + (f & 1)).reshape(NW, NCH, CHUNK)
    e_lin = jax.lax.optimization_barrier(e_table.reshape(V * E))
    emb_flat, wg_flat = _sc_gather(idx_flat, dest3, e_lin.reshape(V, E),
                                   w_table.reshape(V))
    emb2 = emb_flat.reshape(BF // 2, 2 * E)
    wg = wg_flat.reshape(B, F)
    val = feat_val.astype(jnp.float32)

    bias6 = jnp.stack([b0, g0, bt0, b1, g1, bt1], axis=0)
    R = jnp.repeat(jnp.eye(F, dtype=jnp.float32), E, axis=1)

    return _tc_fused(emb2, val, wg, W0.astype(jnp.bfloat16),
                     W1.astype(jnp.bfloat16), bias6, R,
                     Wfc[:F].reshape(1, F), Wfc[F:F + E].reshape(1, E),
                     Wfc[F + E:], bfc.reshape(1, 1))


# speedup vs baseline: 2.2561x; 1.0013x over previous
import functools

import jax
import jax.numpy as jnp
from jax import lax
from jax.experimental import pallas as pl
from jax.experimental.pallas import tpu as pltpu
from jax.experimental.pallas import tpu_sc as plsc

B, F, V, E = 4096, 26, 100000, 64
D_IN = F * E
H = 400
BF = B * F
NW = 32
PERW = BF // NW
CHUNK = 128
NCH = PERW // CHUNK
BB = 512
NB = B // BB
EPS = 1e-3


NS = D_IN // 128


def _sc_gather(idx_flat, dest3, e_table, w_flat):
    mesh = plsc.VectorSubcoreMesh(core_axis_name="c", subcore_axis_name="s")

    @functools.partial(
        pl.kernel,
        out_type=[
            jax.ShapeDtypeStruct((BF, E), jnp.float32),
            jax.ShapeDtypeStruct((BF,), jnp.float32),
        ],
        mesh=mesh,
        compiler_params=pltpu.CompilerParams(use_tc_tiling_on_sc=False,
                                             needs_layout_passes=False),
        scratch_types=[
            pltpu.VMEM((PERW,), jnp.int32),
            pltpu.VMEM((NCH, CHUNK), jnp.int32),
            pltpu.VMEM((CHUNK, E), jnp.float32),
            pltpu.VMEM((CHUNK, E), jnp.float32),
            pltpu.VMEM((PERW,), jnp.float32),
            pltpu.SemaphoreType.DMA,
            pltpu.SemaphoreType.DMA,
            pltpu.SemaphoreType.DMA,
            pltpu.SemaphoreType.DMA,
            pltpu.SemaphoreType.DMA,
        ],
    )
    def k(idx_hbm, dest_hbm, etab_hbm, wtab_hbm, emb_out, w_out,
          idx_v, dest_v, rows0, rows1, wvals, sem0, sem1, ss0, ss1, semw):
        wid = lax.axis_index("s") * 2 + lax.axis_index("c")
        base = wid * PERW
        pltpu.sync_copy(idx_hbm.at[pl.ds(base, PERW)], idx_v)
        pltpu.sync_copy(dest_hbm.at[wid], dest_v)

        def ix(c):
            return idx_v.at[pl.ds(c * CHUNK, CHUNK)]

        def gfire(c, buf, sem):
            pltpu.async_copy(etab_hbm.at[ix(c)], buf, sem)

        def gwait(c, buf, sem):
            pltpu.make_async_copy(etab_hbm.at[ix(c)], buf, sem).wait()

        def sfire(c, buf, sem):
            pltpu.async_copy(buf, emb_out.at[dest_v.at[c]], sem)

        def swait(c, buf, sem):
            pltpu.make_async_copy(buf, emb_out.at[dest_v.at[c]], sem).wait()

        def wfire(c):
            pltpu.async_copy(wtab_hbm.at[ix(c)],
                             wvals.at[pl.ds(c * CHUNK, CHUNK)], semw)

        gfire(0, rows0, sem0)
        wfire(0)
        gfire(1, rows1, sem1)
        wfire(1)
        gwait(0, rows0, sem0)
        sfire(0, rows0, ss0)
        gwait(1, rows1, sem1)
        sfire(1, rows1, ss1)

        @pl.loop(1, NCH // 2)
        def _(g):
            c0 = 2 * g
            c1 = c0 + 1
            swait(c0 - 2, rows0, ss0)
            gfire(c0, rows0, sem0)
            wfire(c0)
            swait(c1 - 2, rows1, ss1)
            gfire(c1, rows1, sem1)
            wfire(c1)
            gwait(c0, rows0, sem0)
            sfire(c0, rows0, ss0)
            gwait(c1, rows1, sem1)
            sfire(c1, rows1, ss1)

        swait(NCH - 2, rows0, ss0)
        swait(NCH - 1, rows1, ss1)

        @pl.loop(0, NCH)
        def _(c):
            pltpu.make_async_copy(wtab_hbm.at[ix(c)],
                                  wvals.at[pl.ds(c * CHUNK, CHUNK)],
                                  semw).wait()

        pltpu.sync_copy(wvals, w_out.at[pl.ds(base, PERW)])

    return k(idx_flat, dest3, e_table, w_flat)


def _fused_body(*refs):
    (e_refs, (val_ref, wg_ref, w0_ref, w1_ref, bias_ref,
              r_ref, wfca_ref, wfcb_ref, wfcc_ref, bfc_ref,
              out_ref, h0_s, h1_s, lin_s, fm_s, st0_s, st1_s)) = \
        refs[:NS], refs[NS:]
    p = pl.program_id(0)
    i = pl.program_id(1)

    @pl.when(p == 0)
    def _():
        val = val_ref[...]
        vexp = jnp.dot(val, r_ref[...], preferred_element_type=jnp.float32)
        h0 = None
        s = None
        s2 = None
        for c in range(NS):
            ec = e_refs[c][...]
            pc = jnp.dot(ec.astype(jnp.bfloat16),
                         w0_ref[pl.ds(c * 128, 128), :],
                         preferred_element_type=jnp.float32)
            t = ec * vexp[:, c * 128:(c + 1) * 128]
            tsq = t * t
            sc_ = t[:, :E] + t[:, E:]
            s2c = tsq[:, :E] + tsq[:, E:]
            h0 = pc if h0 is None else h0 + pc
            s = sc_ if s is None else s + sc_
            s2 = s2c if s2 is None else s2 + s2c
        h0 = h0 + bias_ref[0:1, :H]
        h0_s[pl.ds(i * BB, BB), :] = h0
        lin_s[pl.ds(i * BB, BB), :] = wg_ref[...] * val
        fm_s[pl.ds(i * BB, BB), :] = 0.5 * (s * s - s2)
        ps = jnp.concatenate(
            [jnp.sum(h0, axis=0, keepdims=True),
             jnp.sum(h0 * h0, axis=0, keepdims=True)], axis=0)

        @pl.when(i == 0)
        def _():
            st0_s[...] = jnp.zeros_like(st0_s)

        st0_s[...] += ps

    @pl.when(p == 1)
    def _():
        st = st0_s[...]
        m = st[0:1, :] * (1.0 / B)
        v = st[1:2, :] * (1.0 / B) - m * m
        inv = bias_ref[1:2, :H] * lax.rsqrt(v + EPS)
        h0 = h0_s[pl.ds(i * BB, BB), :]
        a = jnp.maximum((h0 - m) * inv + bias_ref[2:3, :H], 0.0)
        h1 = jnp.dot(a.astype(jnp.bfloat16), w1_ref[...],
                     preferred_element_type=jnp.float32)
        h1 = h1 + bias_ref[3:4, :H]
        h1_s[pl.ds(i * BB, BB), :] = h1
        ps = jnp.concatenate(
            [jnp.sum(h1, axis=0, keepdims=True),
             jnp.sum(h1 * h1, axis=0, keepdims=True)], axis=0)

        @pl.when(i == 0)
        def _():
            st1_s[...] = jnp.zeros_like(st1_s)

        st1_s[...] += ps

    @pl.when(p == 2)
    def _():
        st = st1_s[...]
        m = st[0:1, :] * (1.0 / B)
        v = st[1:2, :] * (1.0 / B) - m * m
        inv = bias_ref[4:5, :H] * lax.rsqrt(v + EPS)
        h1 = h1_s[pl.ds(i * BB, BB), :]
        y = jnp.maximum((h1 - m) * inv + bias_ref[5:6, :H], 0.0)
        logit = (jnp.sum(lin_s[pl.ds(i * BB, BB), :] * wfca_ref[...],
                         axis=1, keepdims=True)
                 + jnp.sum(fm_s[pl.ds(i * BB, BB), :] * wfcb_ref[...],
                           axis=1, keepdims=True)
                 + jnp.dot(y, wfcc_ref[...],
                           preferred_element_type=jnp.float32)
                 + bfc_ref[...])
        out_ref[...] = jax.nn.sigmoid(logit)


def _tc_fused(emb2, val, wg, W0, W1, bias6, R, wfca, wfcb, wfcc, bfc):
    def eb(p, i):
        return (jnp.where(p == 0, i, NB - 1), 0)

    def slab(c):
        return pl.BlockSpec(
            (BB, 128), lambda p, i, c=c: (jnp.where(p == 0, i, NB - 1) + c * NB, 0))

    def cst(p, i):
        return (0, 0)

    return pl.pallas_call(
        _fused_body,
        grid=(3, NB),
        in_specs=[slab(c) for c in range(NS)] + [
            pl.BlockSpec((BB, F), eb),
            pl.BlockSpec((BB, F), eb),
            pl.BlockSpec((D_IN, H), cst),
            pl.BlockSpec((H, H), cst),
            pl.BlockSpec((6, H), cst),
            pl.BlockSpec((F, D_IN), cst),
            pl.BlockSpec((1, F), cst),
            pl.BlockSpec((1, E), cst),
            pl.BlockSpec((H, 1), cst),
            pl.BlockSpec((1, 1), cst),
        ],
        out_specs=pl.BlockSpec((BB, 1), lambda p, i: (jnp.where(p == 2, i, 0), 0)),
        out_shape=jax.ShapeDtypeStruct((B, 1), jnp.float32),
        scratch_shapes=[
            pltpu.VMEM((B, H), jnp.float32),
            pltpu.VMEM((B, H), jnp.float32),
            pltpu.VMEM((B, F), jnp.float32),
            pltpu.VMEM((B, E), jnp.float32),
            pltpu.VMEM((2, H), jnp.float32),
            pltpu.VMEM((2, H), jnp.float32),
        ],
    )(*([emb2] * NS), val, wg, W0, W1, bias6, R, wfca, wfcb, wfcc, bfc)


def kernel(feat_idx, feat_val, w_table, e_table,
           W0, b0, g0, bt0, W1, b1, g1, bt1, Wfc, bfc):
    idx_flat = feat_idx.astype(jnp.int32).reshape(BF)
    j = jnp.arange(BF, dtype=jnp.int32)
    b = j // F
    f = j % F
    dest3 = ((f >> 1) * (B * 2) + (b >> 3) * 16 + (b & 7) * 2
             + (f & 1)).reshape(NW, NCH, CHUNK)
    emb_flat, wg_flat = _sc_gather(idx_flat, dest3, e_table,
                                   w_table.reshape(V))
    emb2 = emb_flat.reshape(BF // 2, 2 * E)
    wg = wg_flat.reshape(B, F)
    val = feat_val.astype(jnp.float32)

    bias6 = jnp.stack([b0, g0, bt0, b1, g1, bt1], axis=0)
    R = jnp.repeat(jnp.eye(F, dtype=jnp.float32), E, axis=1)

    return _tc_fused(emb2, val, wg, W0.astype(jnp.bfloat16),
                     W1.astype(jnp.bfloat16), bias6, R,
                     Wfc[:F].reshape(1, F), Wfc[F:F + E].reshape(1, E),
                     Wfc[F + E:], bfc.reshape(1, 1))
